# baseline (speedup 1.0000x reference)
import functools
import jax
import jax.numpy as jnp
from jax import lax
from jax.experimental import pallas as pl
from jax.experimental.pallas import tpu as pltpu, tpu_sc as plsc

_INFO = plsc.get_sparse_core_info()
_NC, _NS = _INFO.num_cores, _INFO.num_subcores
_NW = _NC * _NS


def kernel(input_ids, table):
    seq_len = input_ids.shape[1]
    rows, dim = table.shape
    rows_per_w = rows // _NW

    mesh = plsc.VectorSubcoreMesh(core_axis_name="c", subcore_axis_name="s")

    @functools.partial(
        pl.kernel,
        mesh=mesh,
        out_type=jax.ShapeDtypeStruct((rows, dim), table.dtype),
        scratch_types=[pltpu.SemaphoreType.DMA],
    )
    def sc_copy(table_hbm, out_hbm, sem):
        wid = lax.axis_index("s") * _NC + lax.axis_index("c")
        base = wid * rows_per_w
        copy = pltpu.make_async_copy(
            table_hbm.at[pl.ds(base, rows_per_w)],
            out_hbm.at[pl.ds(base, rows_per_w)],
            sem,
        )
        copy.start()
        copy.wait()

    out = sc_copy(table)
    return out[None]


# speedup vs baseline: 24.2931x; 24.2931x over previous
import functools
import jax
import jax.numpy as jnp
from jax import lax
from jax.experimental import pallas as pl
from jax.experimental.pallas import tpu as pltpu, tpu_sc as plsc

_INFO = plsc.get_sparse_core_info()
_NC, _NS = _INFO.num_cores, _INFO.num_subcores
_NW = _NC * _NS
_CH = 32


def kernel(input_ids, table):
    seq_len = input_ids.shape[1]
    rows, dim = table.shape
    rows_per_w = rows // _NW
    nchunks = rows_per_w // _CH

    mesh = plsc.VectorSubcoreMesh(core_axis_name="c", subcore_axis_name="s")

    @functools.partial(
        pl.kernel,
        mesh=mesh,
        out_type=jax.ShapeDtypeStruct((rows, dim), table.dtype),
        scratch_types=[
            pltpu.VMEM((2, _CH, dim), table.dtype),
            pltpu.SemaphoreType.DMA,
            pltpu.SemaphoreType.DMA,
            pltpu.SemaphoreType.DMA,
            pltpu.SemaphoreType.DMA,
        ],
    )
    def sc_copy(table_hbm, out_hbm, buf, in0, in1, ou0, ou1):
        wid = lax.axis_index("s") * _NC + lax.axis_index("c")
        base = wid * rows_per_w
        in_sems = (in0, in1)
        out_sems = (ou0, ou1)

        def in_copy(i):
            return pltpu.make_async_copy(
                table_hbm.at[pl.ds(base + i * _CH, _CH)],
                buf.at[i % 2],
                in_sems[i % 2],
            )

        def out_copy(i):
            return pltpu.make_async_copy(
                buf.at[i % 2],
                out_hbm.at[pl.ds(base + i * _CH, _CH)],
                out_sems[i % 2],
            )

        for i in range(nchunks):
            if i >= 2:
                out_copy(i - 2).wait()
            in_copy(i).start()
            if i >= 1:
                in_copy(i - 1).wait()
                out_copy(i - 1).start()
        in_copy(nchunks - 1).wait()
        out_copy(nchunks - 1).start()
        out_copy(nchunks - 2).wait()
        out_copy(nchunks - 1).wait()

    out = sc_copy(table)
    return out[None]
